# Initial kernel scaffold; baseline (speedup 1.0000x reference)
import jax
import jax.numpy as jnp
from jax.experimental import pallas as pl


def kernel(x, boundaries, data_type):
    raise NotImplementedError("write your pallas kernel here")


# speedup vs baseline: 9.4631x; 9.4631x over previous
import jax
import jax.numpy as jnp
from jax.experimental import pallas as pl
from jax.experimental.pallas import tpu as pltpu

_ROWS = 8192
_COLS = 8192
_BLK_ROWS = 256


def _tc_body(b_ref, dt_ref, x_ref, o_ref):
    x = x_ref[...]
    s = jnp.max(jnp.abs(x), axis=1, keepdims=True)
    s = jnp.maximum(s, 1e-6)
    val = jnp.full(x.shape, dt_ref[0], jnp.float32)
    for i in range(15):
        t = b_ref[i] * s
        val = jnp.where(x > t, dt_ref[i + 1], val)
    o_ref[...] = val * s


def kernel(x, boundaries, data_type):
    grid = (_ROWS // _BLK_ROWS,)
    return pl.pallas_call(
        _tc_body,
        grid=grid,
        in_specs=[
            pl.BlockSpec(memory_space=pltpu.SMEM),
            pl.BlockSpec(memory_space=pltpu.SMEM),
            pl.BlockSpec((_BLK_ROWS, _COLS), lambda i: (i, 0)),
        ],
        out_specs=pl.BlockSpec((_BLK_ROWS, _COLS), lambda i: (i, 0)),
        out_shape=jax.ShapeDtypeStruct((_ROWS, _COLS), jnp.float32),
    )(boundaries, data_type, x)
